# Initial kernel scaffold; baseline (speedup 1.0000x reference)
import jax
import jax.numpy as jnp
from jax.experimental import pallas as pl


def kernel(input_ids, attention_mask, labels, embed_table, prompt_embed):
    raise NotImplementedError("write your pallas kernel here")


# speedup vs baseline: 2.2651x; 2.2651x over previous
import functools

import jax
import jax.numpy as jnp
from jax import lax
from jax.experimental import pallas as pl
from jax.experimental.pallas import tpu as pltpu
from jax.experimental.pallas import tpu_sc as plsc

_P = 20
_K = 128
_CH = 32


def _embed_call(ids_w, embed_table, prompt_embed, B, NB, D, new_len):
    info = plsc.get_sparse_core_info()
    NC, NS = info.num_cores, info.num_subcores
    NW = NC * NS
    units = B * NB
    upw = units // NW
    cpu_ = _K // _CH
    n_chunks = upw * cpu_

    mesh = plsc.VectorSubcoreMesh(core_axis_name="c", subcore_axis_name="s")

    @functools.partial(
        pl.kernel,
        out_type=jax.ShapeDtypeStruct((B, new_len, D), jnp.float32),
        mesh=mesh,
        scratch_types=[
            pltpu.VMEM((n_chunks, _CH), jnp.int32),
            pltpu.VMEM((_P, D), jnp.float32),
            pltpu.VMEM((_CH, D), jnp.float32),
            pltpu.SemaphoreType.DMA,
        ],
        compiler_params=pltpu.CompilerParams(use_tc_tiling_on_sc=False),
    )
    def sc_embed(ids_hbm, table_hbm, prompt_hbm, out_hbm,
                 idx_v, prompt_v, rows_v, gsem):
        wid = lax.axis_index("s") * NC + lax.axis_index("c")
        pltpu.sync_copy(ids_hbm.at[wid], idx_v)
        pltpu.sync_copy(prompt_hbm, prompt_v)
        for k in range(upw):
            u = wid * upw + k
            b = u // NB
            n = u % NB
            pltpu.sync_copy(prompt_v, out_hbm.at[b, pl.ds(n * (_K + _P), _P), :])
            for c in range(cpu_):
                i = k * cpu_ + c
                pltpu.async_copy(table_hbm.at[idx_v.at[i]], rows_v, gsem).wait()
                row0 = n * (_K + _P) + _P + c * _CH
                pltpu.sync_copy(rows_v, out_hbm.at[b, pl.ds(row0, _CH), :])

    return sc_embed(ids_w, embed_table, prompt_embed)


def _mask_labels_call(am3, lab3, B, NB):
    def body(am_ref, lab_ref, mask_ref, labout_ref):
        ones = jnp.ones((B, NB, _P), jnp.int32)
        mask_ref[...] = jnp.concatenate([ones, am_ref[...]], axis=2)
        neg = jnp.full((B, NB, _P), -100, jnp.int32)
        labout_ref[...] = jnp.concatenate([neg, lab_ref[...]], axis=2)

    out_sd = jax.ShapeDtypeStruct((B, NB, _P + _K), jnp.int32)
    return pl.pallas_call(body, out_shape=(out_sd, out_sd))(am3, lab3)


def kernel(input_ids, attention_mask, labels, embed_table, prompt_embed):
    B, T = input_ids.shape
    V, D = embed_table.shape
    NB = T // _K
    new_len = NB * (_K + _P)

    ids_w = input_ids.reshape(32, (B * T) // (32 * _CH), _CH)
    out = _embed_call(ids_w, embed_table, prompt_embed, B, NB, D, new_len)

    am3 = attention_mask.reshape(B, NB, _K)
    lab3 = labels.reshape(B, NB, _K)
    mask3, lab_out3 = _mask_labels_call(am3, lab3, B, NB)
    return out, mask3.reshape(B, new_len), lab_out3.reshape(B, new_len)


# speedup vs baseline: 2.3297x; 1.0285x over previous
import functools

import jax
import jax.numpy as jnp
from jax import lax
from jax.experimental import pallas as pl
from jax.experimental.pallas import tpu as pltpu
from jax.experimental.pallas import tpu_sc as plsc

_P = 20
_K = 128
_CH = 32
_NBUF = 3


def _embed_call(ids_w, embed_table, prompt_embed, B, NB, D, new_len):
    info = plsc.get_sparse_core_info()
    NC, NS = info.num_cores, info.num_subcores
    NW = NC * NS
    units = B * NB
    upw = units // NW
    cpu_ = _K // _CH
    n_chunks = upw * cpu_

    mesh = plsc.VectorSubcoreMesh(core_axis_name="c", subcore_axis_name="s")

    @functools.partial(
        pl.kernel,
        out_type=jax.ShapeDtypeStruct((B, new_len, D), jnp.float32),
        mesh=mesh,
        scratch_types=[
            pltpu.VMEM((n_chunks, _CH), jnp.int32),
            pltpu.VMEM((_P, D), jnp.float32),
            pltpu.VMEM((_NBUF, _CH, D), jnp.float32),
            pltpu.SemaphoreType.DMA((_NBUF,)),
            pltpu.SemaphoreType.DMA((_NBUF,)),
            pltpu.SemaphoreType.DMA,
        ],
        compiler_params=pltpu.CompilerParams(use_tc_tiling_on_sc=False),
    )
    def sc_embed(ids_hbm, table_hbm, prompt_hbm, out_hbm,
                 idx_v, prompt_v, rows_v, gsem, osem, psem):
        wid = lax.axis_index("s") * NC + lax.axis_index("c")
        pltpu.sync_copy(ids_hbm.at[wid], idx_v)
        pltpu.sync_copy(prompt_hbm, prompt_v)

        def unit_bn(i):
            u = wid * upw + (i // cpu_)
            return u // NB, u % NB

        pdescs = []
        for k in range(upw):
            b, n = unit_bn(k * cpu_)
            pdescs.append(pltpu.async_copy(
                prompt_v, out_hbm.at[b, pl.ds(n * (_K + _P), _P), :], psem))

        def start_g(i):
            return pltpu.async_copy(
                table_hbm.at[idx_v.at[i]], rows_v.at[i % _NBUF],
                gsem.at[i % _NBUF])

        def start_o(i):
            b, n = unit_bn(i)
            row0 = n * (_K + _P) + _P + (i % cpu_) * _CH
            return pltpu.async_copy(
                rows_v.at[i % _NBUF], out_hbm.at[b, pl.ds(row0, _CH), :],
                osem.at[i % _NBUF])

        gd = {i: start_g(i) for i in range(min(_NBUF, n_chunks))}
        od = {}
        for i in range(n_chunks):
            gd[i].wait()
            od[i] = start_o(i)
            if i + _NBUF < n_chunks:
                od[i].wait()
                gd[i + _NBUF] = start_g(i + _NBUF)
        for i in range(max(0, n_chunks - _NBUF), n_chunks):
            od[i].wait()
        for d in pdescs:
            d.wait()

    return sc_embed(ids_w, embed_table, prompt_embed)


def _mask_labels_call(am3, lab3, B, NB):
    def body(am_ref, lab_ref, mask_ref, labout_ref):
        ones = jnp.ones((B, NB, _P), jnp.int32)
        mask_ref[...] = jnp.concatenate([ones, am_ref[...]], axis=2)
        neg = jnp.full((B, NB, _P), -100, jnp.int32)
        labout_ref[...] = jnp.concatenate([neg, lab_ref[...]], axis=2)

    out_sd = jax.ShapeDtypeStruct((B, NB, _P + _K), jnp.int32)
    return pl.pallas_call(body, out_shape=(out_sd, out_sd))(am3, lab3)


def kernel(input_ids, attention_mask, labels, embed_table, prompt_embed):
    B, T = input_ids.shape
    V, D = embed_table.shape
    NB = T // _K
    new_len = NB * (_K + _P)

    ids_w = input_ids.reshape(32, (B * T) // (32 * _CH), _CH)
    out = _embed_call(ids_w, embed_table, prompt_embed, B, NB, D, new_len)

    am3 = attention_mask.reshape(B, NB, _K)
    lab3 = labels.reshape(B, NB, _K)
    mask3, lab_out3 = _mask_labels_call(am3, lab3, B, NB)
    return out, mask3.reshape(B, new_len), lab_out3.reshape(B, new_len)


# speedup vs baseline: 5.3175x; 2.2825x over previous
import functools

import jax
import jax.numpy as jnp
from jax import lax
from jax.experimental import pallas as pl
from jax.experimental.pallas import tpu as pltpu
from jax.experimental.pallas import tpu_sc as plsc

_P = 20
_K = 128
_CH = 32
_NBUF = 3


def _gather_call(ids_w, embed_table, B, T, D):
    info = plsc.get_sparse_core_info()
    NC, NS = info.num_cores, info.num_subcores
    NW = NC * NS
    rows_pw = (B * T) // NW
    n_chunks = rows_pw // _CH

    mesh = plsc.VectorSubcoreMesh(core_axis_name="c", subcore_axis_name="s")

    @functools.partial(
        pl.kernel,
        out_type=jax.ShapeDtypeStruct((B * T, D), jnp.float32),
        mesh=mesh,
        scratch_types=[
            pltpu.VMEM((n_chunks, _CH), jnp.int32),
            pltpu.VMEM((_NBUF, _CH, D), jnp.float32),
            pltpu.SemaphoreType.DMA((_NBUF,)),
            pltpu.SemaphoreType.DMA((_NBUF,)),
        ],
        compiler_params=pltpu.CompilerParams(use_tc_tiling_on_sc=True),
    )
    def sc_gather(ids_hbm, table_hbm, out_hbm, idx_v, rows_v, gsem, osem):
        wid = lax.axis_index("s") * NC + lax.axis_index("c")
        pltpu.sync_copy(ids_hbm.at[wid], idx_v)
        base = wid * rows_pw

        def start_g(i):
            return pltpu.async_copy(
                table_hbm.at[idx_v.at[i]], rows_v.at[i % _NBUF],
                gsem.at[i % _NBUF])

        def start_o(i):
            return pltpu.async_copy(
                rows_v.at[i % _NBUF],
                out_hbm.at[pl.ds(base + i * _CH, _CH), :],
                osem.at[i % _NBUF])

        gd = {i: start_g(i) for i in range(min(_NBUF, n_chunks))}
        od = {}
        for i in range(n_chunks):
            gd[i].wait()
            od[i] = start_o(i)
            if i + _NBUF < n_chunks:
                od[i].wait()
                gd[i + _NBUF] = start_g(i + _NBUF)
        for i in range(max(0, n_chunks - _NBUF), n_chunks):
            od[i].wait()

    return sc_gather(ids_w, embed_table)


_BPG = 2


def _assemble_call(x3, prompt_embed, B, NB, D, new_len):
    W = _P + _K

    def body(x_ref, p_ref, out_ref):
        for j in range(_BPG):
            out_ref[0, j * W:j * W + _P, :] = p_ref[...]
            out_ref[0, j * W + _P:(j + 1) * W, :] = x_ref[0, j * _K:(j + 1) * _K, :]

    return pl.pallas_call(
        body,
        grid=(B, NB // _BPG),
        in_specs=[
            pl.BlockSpec((1, _BPG * _K, D), lambda b, h: (b, h, 0)),
            pl.BlockSpec((_P, D), lambda b, h: (0, 0)),
        ],
        out_specs=pl.BlockSpec((1, _BPG * W, D), lambda b, h: (b, h, 0)),
        out_shape=jax.ShapeDtypeStruct((B, new_len, D), jnp.float32),
    )(x3, prompt_embed)


def _mask_labels_call(am3, lab3, B, NB):
    def body(am_ref, lab_ref, mask_ref, labout_ref):
        mask_ref[...] = jnp.concatenate(
            [jnp.ones((B, NB, 1, _P), jnp.int32), am_ref[...]], axis=3)
        labout_ref[...] = jnp.concatenate(
            [jnp.full((B, NB, 1, _P), -100, jnp.int32), lab_ref[...]], axis=3)

    out_sd = jax.ShapeDtypeStruct((B, NB, 1, _P + _K), jnp.int32)
    return pl.pallas_call(body, out_shape=(out_sd, out_sd))(am3, lab3)


def kernel(input_ids, attention_mask, labels, embed_table, prompt_embed):
    B, T = input_ids.shape
    V, D = embed_table.shape
    NB = T // _K
    new_len = NB * (_K + _P)

    ids_w = input_ids.reshape(32, (B * T) // (32 * _CH), _CH)
    x = _gather_call(ids_w, embed_table, B, T, D)

    out = _assemble_call(x.reshape(B, T, D), prompt_embed, B, NB, D, new_len)

    am3 = attention_mask.reshape(B, NB, 1, _K)
    lab3 = labels.reshape(B, NB, 1, _K)
    mask3, lab_out3 = _mask_labels_call(am3, lab3, B, NB)
    return out, mask3.reshape(B, new_len), lab_out3.reshape(B, new_len)


# speedup vs baseline: 5.5865x; 1.0506x over previous
import functools

import jax
import jax.numpy as jnp
from jax import lax
from jax.experimental import pallas as pl
from jax.experimental.pallas import tpu as pltpu
from jax.experimental.pallas import tpu_sc as plsc

_P = 20
_K = 128
_CH = 32
_NBUF = 3
_BPG = 2


def _gather_call(ids_w, embed_table, n_rows, D):
    info = plsc.get_sparse_core_info()
    NC, NS = info.num_cores, info.num_subcores
    NW = NC * NS
    rows_pw = n_rows // NW
    n_chunks = rows_pw // _CH

    mesh = plsc.VectorSubcoreMesh(core_axis_name="c", subcore_axis_name="s")

    @functools.partial(
        pl.kernel,
        out_type=jax.ShapeDtypeStruct((n_rows, D), jnp.float32),
        mesh=mesh,
        scratch_types=[
            pltpu.VMEM((n_chunks, _CH), jnp.int32),
            pltpu.VMEM((_NBUF, _CH, D), jnp.float32),
            pltpu.SemaphoreType.DMA((_NBUF,)),
            pltpu.SemaphoreType.DMA((_NBUF,)),
        ],
        compiler_params=pltpu.CompilerParams(use_tc_tiling_on_sc=True),
    )
    def sc_gather(ids_hbm, table_hbm, out_hbm, idx_v, rows_v, gsem, osem):
        wid = lax.axis_index("s") * NC + lax.axis_index("c")
        pltpu.sync_copy(ids_hbm.at[wid], idx_v)
        base = wid * rows_pw

        def start_g(i):
            return pltpu.async_copy(
                table_hbm.at[idx_v.at[i]], rows_v.at[i % _NBUF],
                gsem.at[i % _NBUF])

        def start_o(i):
            return pltpu.async_copy(
                rows_v.at[i % _NBUF],
                out_hbm.at[pl.ds(base + i * _CH, _CH), :],
                osem.at[i % _NBUF])

        gd = {i: start_g(i) for i in range(min(_NBUF, n_chunks))}
        od = {}
        for i in range(n_chunks):
            gd[i].wait()
            od[i] = start_o(i)
            if i + _NBUF < n_chunks:
                od[i].wait()
                gd[i + _NBUF] = start_g(i + _NBUF)
        for i in range(max(0, n_chunks - _NBUF), n_chunks):
            od[i].wait()

    return sc_gather(ids_w, embed_table)


def _assemble_half(xh, prompt_embed, out_buf, b0, B, Bh, NB, D, new_len):
    W = _P + _K

    def body(x_ref, p_ref, *rest):
        out_ref = rest[-1]
        for j in range(_BPG):
            out_ref[0, j * W:j * W + _P, :] = p_ref[...]
            out_ref[0, j * W + _P:(j + 1) * W, :] = x_ref[0, j * _K:(j + 1) * _K, :]

    in_specs = [
        pl.BlockSpec((1, _BPG * _K, D), lambda b, h: (b, h, 0)),
        pl.BlockSpec((_P, D), lambda b, h: (0, 0)),
    ]
    args = [xh, prompt_embed]
    aliases = {}
    if out_buf is not None:
        in_specs.append(pl.BlockSpec(memory_space=pl.ANY))
        args.append(out_buf)
        aliases = {2: 0}
    return pl.pallas_call(
        body,
        grid=(Bh, NB // _BPG),
        in_specs=in_specs,
        out_specs=pl.BlockSpec((1, _BPG * W, D),
                               lambda b, h, b0=b0: (b + b0, h, 0)),
        out_shape=jax.ShapeDtypeStruct((B, new_len, D), jnp.float32),
        input_output_aliases=aliases,
    )(*args)


def _mask_labels_call(am3, lab3, B, NB):
    def body(am_ref, lab_ref, mask_ref, labout_ref):
        mask_ref[...] = jnp.concatenate(
            [jnp.ones((B, NB, 1, _P), jnp.int32), am_ref[...]], axis=3)
        labout_ref[...] = jnp.concatenate(
            [jnp.full((B, NB, 1, _P), -100, jnp.int32), lab_ref[...]], axis=3)

    out_sd = jax.ShapeDtypeStruct((B, NB, 1, _P + _K), jnp.int32)
    return pl.pallas_call(body, out_shape=(out_sd, out_sd))(am3, lab3)


def kernel(input_ids, attention_mask, labels, embed_table, prompt_embed):
    B, T = input_ids.shape
    V, D = embed_table.shape
    NB = T // _K
    new_len = NB * (_K + _P)
    Bh = B // 2

    ids_h = input_ids.reshape(2, 32, (Bh * T) // (32 * _CH), _CH)
    x0 = _gather_call(ids_h[0], embed_table, Bh * T, D)
    x1 = _gather_call(ids_h[1], embed_table, Bh * T, D)

    out_buf = _assemble_half(x0.reshape(Bh, T, D), prompt_embed, None,
                             0, B, Bh, NB, D, new_len)
    out = _assemble_half(x1.reshape(Bh, T, D), prompt_embed, out_buf,
                         Bh, B, Bh, NB, D, new_len)

    am3 = attention_mask.reshape(B, NB, 1, _K)
    lab3 = labels.reshape(B, NB, 1, _K)
    mask3, lab_out3 = _mask_labels_call(am3, lab3, B, NB)
    return out, mask3.reshape(B, new_len), lab_out3.reshape(B, new_len)


# speedup vs baseline: 8.1026x; 1.4504x over previous
import functools

import jax
import jax.numpy as jnp
from jax import lax
from jax.experimental import pallas as pl
from jax.experimental.pallas import tpu as pltpu
from jax.experimental.pallas import tpu_sc as plsc

_P = 20
_K = 128
_CH = 32
_NBUF = 3
_W2 = 2 * (_P + _K)

_PLAN = (
    (0, 24, 32), (1, 56, 32), (2, 88, 32), (3, 120, 24),
    (4, 168, 32), (5, 200, 32), (6, 232, 32), (7, 264, 32),
)


def _gather_call(idx_chunks, fringe_idx, embed_table, B, T, D, new_len):
    info = plsc.get_sparse_core_info()
    NC, NS = info.num_cores, info.num_subcores
    NW = NC * NS

    mesh = plsc.VectorSubcoreMesh(core_axis_name="c", subcore_axis_name="s")

    @functools.partial(
        pl.kernel,
        out_type=[
            jax.ShapeDtypeStruct((B * new_len, D), jnp.float32),
            jax.ShapeDtypeStruct((NW, 16, D), jnp.float32),
        ],
        mesh=mesh,
        scratch_types=[
            pltpu.VMEM((8, _CH), jnp.int32),
            pltpu.VMEM((16,), jnp.int32),
            pltpu.VMEM((_NBUF, _CH, D), jnp.float32),
            pltpu.VMEM((16, D), jnp.float32),
            pltpu.SemaphoreType.DMA((_NBUF,)),
            pltpu.SemaphoreType.DMA((_NBUF,)),
            pltpu.SemaphoreType.DMA,
        ],
        compiler_params=pltpu.CompilerParams(use_tc_tiling_on_sc=True),
    )
    def sc_gather(idx_hbm, fr_hbm, table_hbm, out_hbm, side_hbm,
                  idx_v, fr_v, rows_v, frows_v, gsem, osem, fsem):
        wid = lax.axis_index("s") * NC + lax.axis_index("c")
        pltpu.sync_copy(idx_hbm.at[wid], idx_v)
        pltpu.sync_copy(fr_hbm.at[wid], fr_v)
        base = wid * _W2

        fg = pltpu.async_copy(table_hbm.at[fr_v], frows_v, fsem)

        def start_g(i):
            return pltpu.async_copy(
                table_hbm.at[idx_v.at[i]], rows_v.at[i % _NBUF],
                gsem.at[i % _NBUF])

        def start_o(i):
            _, off, rows = _PLAN[i]
            return pltpu.async_copy(
                rows_v.at[i % _NBUF, pl.ds(0, rows)],
                out_hbm.at[pl.ds(base + off, rows), :],
                osem.at[i % _NBUF])

        n_chunks = len(_PLAN)
        gd = {i: start_g(i) for i in range(min(_NBUF, n_chunks))}
        od = {}
        for i in range(n_chunks):
            gd[i].wait()
            od[i] = start_o(i)
            if i + _NBUF < n_chunks:
                od[i].wait()
                gd[i + _NBUF] = start_g(i + _NBUF)
        fg.wait()
        fs = pltpu.async_copy(frows_v, side_hbm.at[wid], fsem)
        for i in range(max(0, n_chunks - _NBUF), n_chunks):
            od[i].wait()
        fs.wait()

    return sc_gather(idx_chunks, fringe_idx, embed_table)


def _finish_call(out_buf, side, prompt_embed, NW, D):
    def body(side_ref, p_ref, _, out_ref, stage_v, sem):
        stage_v[:, 0:_P, :] = jnp.broadcast_to(
            p_ref[...][None], (NW, _P, D))
        stage_v[:, _P:36, :] = side_ref[...]
        stage_v[:, 36:56, :] = jnp.broadcast_to(
            p_ref[...][None], (NW, _P, D))
        copies = []
        for m in range(NW):
            copies.append(pltpu.make_async_copy(
                stage_v.at[m, pl.ds(0, 24)],
                out_ref.at[pl.ds(m * _W2, 24), :], sem))
            copies.append(pltpu.make_async_copy(
                stage_v.at[m, pl.ds(24, 32)],
                out_ref.at[pl.ds(m * _W2 + 136, 32), :], sem))
        for c in copies:
            c.start()
        for c in copies:
            c.wait()

    n_rows = out_buf.shape[0]
    return pl.pallas_call(
        body,
        in_specs=[
            pl.BlockSpec((NW, 16, D), lambda: (0, 0, 0)),
            pl.BlockSpec((_P, D), lambda: (0, 0)),
            pl.BlockSpec(memory_space=pl.ANY),
        ],
        out_specs=pl.BlockSpec(memory_space=pl.ANY),
        out_shape=jax.ShapeDtypeStruct((n_rows, D), jnp.float32),
        input_output_aliases={2: 0},
        scratch_shapes=[
            pltpu.VMEM((NW, 56, D), jnp.float32),
            pltpu.SemaphoreType.DMA,
        ],
    )(side, prompt_embed, out_buf)


def _mask_labels_call(am3, lab3, B, NB):
    def body(am_ref, lab_ref, mask_ref, labout_ref):
        mask_ref[...] = jnp.concatenate(
            [jnp.ones((B, NB, 1, _P), jnp.int32), am_ref[...]], axis=3)
        labout_ref[...] = jnp.concatenate(
            [jnp.full((B, NB, 1, _P), -100, jnp.int32), lab_ref[...]], axis=3)

    out_sd = jax.ShapeDtypeStruct((B, NB, 1, _P + _K), jnp.int32)
    return pl.pallas_call(body, out_shape=(out_sd, out_sd))(am3, lab3)


def kernel(input_ids, attention_mask, labels, embed_table, prompt_embed):
    B, T = input_ids.shape
    V, D = embed_table.shape
    NB = T // _K
    new_len = NB * (_K + _P)
    NW = 32

    ids2 = input_ids.reshape(NW, 2, _K)
    even, odd = ids2[:, 0], ids2[:, 1]
    ec3 = jnp.stack(
        [even[:, 4:36], even[:, 36:68], even[:, 68:100],
         jnp.concatenate([even[:, 100:124], even[:, 120:128]], axis=1)],
        axis=1)
    oc3 = odd.reshape(NW, 4, _CH)
    idx_chunks = jnp.concatenate([ec3, oc3], axis=1)
    fringe_idx = jnp.concatenate([even[:, 0:4], even[:, 116:128]], axis=1)

    out2, side = _gather_call(idx_chunks, fringe_idx, embed_table,
                              B, T, D, new_len)
    out2 = _finish_call(out2, side, prompt_embed, NW, D)
    out = out2.reshape(B, new_len, D)

    am3 = attention_mask.reshape(B, NB, 1, _K)
    lab3 = labels.reshape(B, NB, 1, _K)
    mask3, lab_out3 = _mask_labels_call(am3, lab3, B, NB)
    return out, mask3.reshape(B, new_len), lab_out3.reshape(B, new_len)
